# baseline (speedup 1.0000x reference)
import functools
import jax
import jax.numpy as jnp
from jax import lax
from jax.experimental import pallas as pl
from jax.experimental.pallas import tpu as pltpu
from jax.experimental.pallas import tpu_sc as plsc

_S = 8192
_H = 1024

_TCB = 3
_CHUNK = 1024


def _tc_body(pos_hbm, out_hbm, vmem, in_sem, out_sem):
    n = pos_hbm.shape[0] // _CHUNK

    def in_copy(c):
        return pltpu.make_async_copy(
            pos_hbm.at[pl.ds(c * _CHUNK, _CHUNK), :], vmem.at[c], in_sem.at[c]
        )

    def out_copy(c, b):
        return pltpu.make_async_copy(
            vmem.at[c], out_hbm.at[b, pl.ds(c * _CHUNK, _CHUNK), :], out_sem.at[c]
        )

    for c in range(n):
        in_copy(c).start()
    for c in range(n):
        in_copy(c).wait()
        for b in range(_TCB):
            out_copy(c, b).start()
    for c in range(n):
        for b in range(_TCB):
            out_copy(c, b).wait()


def _tc_part(pos_embedding):
    n = _S // _CHUNK
    return pl.pallas_call(
        _tc_body,
        in_specs=[pl.BlockSpec(memory_space=pl.ANY)],
        out_specs=pl.BlockSpec(memory_space=pl.ANY),
        out_shape=jax.ShapeDtypeStruct((_TCB, _S, _H), pos_embedding.dtype),
        scratch_shapes=[
            pltpu.VMEM((n, _CHUNK, _H), pos_embedding.dtype),
            pltpu.SemaphoreType.DMA((n,)),
            pltpu.SemaphoreType.DMA((n,)),
        ],
    )(pos_embedding)


_NW = 32
_CH = 32
_RPW = _S // _NW
_NCH = _RPW // _CH


def _sc_body(table_hbm, out_hbm, buf, in_sem, out_sem):
    wid = lax.axis_index("s") * 2 + lax.axis_index("c")
    base = wid * _RPW

    def in_copy(c):
        return pltpu.make_async_copy(
            table_hbm.at[pl.ds(base + c * _CH, _CH), :], buf.at[c % 2], in_sem
        )

    def out_copy(c):
        return pltpu.make_async_copy(
            buf.at[c % 2], out_hbm.at[0, pl.ds(base + c * _CH, _CH), :], out_sem
        )

    in_copy(0).start()
    for c in range(_NCH):
        if c + 1 < _NCH:
            if c >= 1:
                out_copy(c - 1).wait()
            in_copy(c + 1).start()
        in_copy(c).wait()
        out_copy(c).start()
    for c in (_NCH - 2, _NCH - 1):
        out_copy(c).wait()


def _sc_part(pos_embedding):
    mesh = plsc.VectorSubcoreMesh(core_axis_name="c", subcore_axis_name="s")
    k = functools.partial(
        pl.kernel,
        mesh=mesh,
        out_type=jax.ShapeDtypeStruct((1, _S, _H), jnp.float32),
        scratch_types=[
            pltpu.VMEM((2, _CH, _H), jnp.float32),
            pltpu.SemaphoreType.DMA,
            pltpu.SemaphoreType.DMA,
        ],
    )(_sc_body)
    return k(pos_embedding)


def kernel(x, pos_embedding):
    tc = _tc_part(pos_embedding)
    sc = _sc_part(pos_embedding)
    return jnp.concatenate([tc, sc], axis=0)


# speedup vs baseline: 2.1904x; 2.1904x over previous
import functools
import jax
import jax.numpy as jnp
from jax import lax
from jax.experimental import pallas as pl
from jax.experimental.pallas import tpu as pltpu
from jax.experimental.pallas import tpu_sc as plsc

_NW = 32
_CH = 32
_NSLOTS = 3


def _sc_body(batch, n_chunks, rows_per_worker, table_hbm, out_hbm, buf,
             in_sem, out_sem):
    wid = lax.axis_index("s") * 2 + lax.axis_index("c")
    base = wid * rows_per_worker

    def in_copy(c):
        return pltpu.make_async_copy(
            table_hbm.at[pl.ds(base + c * _CH, _CH), :],
            buf.at[c % _NSLOTS], in_sem,
        )

    def out_copy(c, b):
        return pltpu.make_async_copy(
            buf.at[c % _NSLOTS],
            out_hbm.at[b, pl.ds(base + c * _CH, _CH), :], out_sem,
        )

    for c in range(min(_NSLOTS, n_chunks)):
        in_copy(c).start()
    for c in range(n_chunks):
        in_copy(c).wait()
        for b in range(batch):
            out_copy(c, b).start()
        nxt = c + _NSLOTS
        if nxt < n_chunks:
            for b in range(batch):
                out_copy(c, b).wait()
            in_copy(nxt).start()
    for c in range(max(0, n_chunks - _NSLOTS), n_chunks):
        for b in range(batch):
            out_copy(c, b).wait()


def kernel(x, pos_embedding):
    B, S = x.shape
    H = pos_embedding.shape[1]
    rows_per_worker = S // _NW
    n_chunks = rows_per_worker // _CH
    mesh = plsc.VectorSubcoreMesh(core_axis_name="c", subcore_axis_name="s")
    body = functools.partial(_sc_body, B, n_chunks, rows_per_worker)
    k = functools.partial(
        pl.kernel,
        mesh=mesh,
        out_type=jax.ShapeDtypeStruct((B, S, H), pos_embedding.dtype),
        scratch_types=[
            pltpu.VMEM((_NSLOTS, _CH, H), pos_embedding.dtype),
            pltpu.SemaphoreType.DMA,
            pltpu.SemaphoreType.DMA,
        ],
    )(body)
    return k(pos_embedding)
